# baseline (speedup 1.0000x reference)
import functools

import jax
import jax.numpy as jnp
from jax import lax
from jax.experimental import pallas as pl
from jax.experimental.pallas import tpu as pltpu
from jax.experimental.pallas import tpu_sc as plsc

N = 100000
D = 128
G = 1024
B = 8
H = 128
HEAD_DIM = 1

NC = 2
NS = 16
NW = NC * NS
CH = 256
NFULL = N // CH
TAILA = 128
TAILB = 32
ROWS_PER_TILE = G // NS


def _pool_body(x_hbm, batch_hbm, sums_hbm,
               buf0, buf1, idx0, idx1, zbuf, tbufa, tbufb, tidxa, tidxb,
               acc_sh, sem0, sem1, isem0, isem1):
    c = lax.axis_index("c")
    s = lax.axis_index("s")
    wid = s * NC + c

    def _fill(r, _):
        for k in range(D // 16):
            zbuf[r, pl.ds(k * 16, 16)] = jnp.zeros((16,), jnp.float32)
        return 0

    lax.fori_loop(0, ROWS_PER_TILE, _fill, 0)

    pltpu.sync_copy(zbuf, acc_sh.at[pl.ds(s * ROWS_PER_TILE, ROWS_PER_TILE)])
    plsc.subcore_barrier()

    nj = (NFULL - wid + NW - 1) // NW
    bufs = ((buf0, idx0, sem0, isem0), (buf1, idx1, sem1, isem1))

    def _issue(j, bufp, idxp, semp, isemp):
        base = (wid + j * NW) * CH
        pltpu.async_copy(x_hbm.at[pl.ds(base, CH)], bufp, semp)
        pltpu.async_copy(batch_hbm.at[pl.ds(base, 128)], idxp.at[0], isemp)
        pltpu.async_copy(batch_hbm.at[pl.ds(base + 128, 128)], idxp.at[1],
                         isemp)

    @pl.when(nj > 0)
    def _prologue():
        _issue(0, buf0, idx0, sem0, isem0)

    def _outer(j2, _):
        for b2 in (0, 1):
            j = j2 * 2 + b2
            bufp, idxp, semp, isemp = bufs[b2]
            bufn, idxn, semn, isemn = bufs[1 - b2]

            @pl.when(j < nj)
            def _do():
                @pl.when(j + 1 < nj)
                def _next():
                    _issue(j + 1, bufn, idxn, semn, isemn)

                base = (wid + j * NW) * CH
                pltpu.make_async_copy(x_hbm.at[pl.ds(base, CH)], bufp,
                                      semp).wait()
                pltpu.make_async_copy(batch_hbm.at[pl.ds(base, 128)],
                                      idxp.at[0], isemp).wait()
                pltpu.make_async_copy(batch_hbm.at[pl.ds(base, 128)],
                                      idxp.at[1], isemp).wait()
                for h in (0, 1):
                    pltpu.sync_copy(bufp.at[pl.ds(h * 128, 128)],
                                    acc_sh.at[idxp.at[h]], add=True)

        return 0

    lax.fori_loop(0, (nj + 1) // 2, _outer, 0)

    @pl.when(wid == NW - 1)
    def _tail():
        base = NFULL * CH
        pltpu.sync_copy(batch_hbm.at[pl.ds(base, TAILA)], tidxa.at[0])
        pltpu.sync_copy(x_hbm.at[pl.ds(base, TAILA)], tbufa)
        pltpu.sync_copy(tbufa, acc_sh.at[tidxa.at[0]], add=True)
        pltpu.sync_copy(batch_hbm.at[pl.ds(base + TAILA, TAILB)], tidxb.at[0])
        pltpu.sync_copy(x_hbm.at[pl.ds(base + TAILA, TAILB)], tbufb)
        pltpu.sync_copy(tbufb, acc_sh.at[tidxb.at[0]], add=True)

    plsc.subcore_barrier()

    pltpu.sync_copy(acc_sh.at[pl.ds(s * ROWS_PER_TILE, ROWS_PER_TILE)],
                    sums_hbm.at[c, pl.ds(s * ROWS_PER_TILE, ROWS_PER_TILE)])


@functools.lru_cache(maxsize=1)
def _get_pool():
  return pl.kernel(
    _pool_body,
    out_type=jax.ShapeDtypeStruct((NC, G, D), jnp.float32),
    mesh=plsc.VectorSubcoreMesh(core_axis_name="c", subcore_axis_name="s",
                                num_cores=NC, num_subcores=NS),
    scratch_types=[
        pltpu.VMEM((CH, D), jnp.float32),
        pltpu.VMEM((CH, D), jnp.float32),
        pltpu.VMEM((2, 128), jnp.int32),
        pltpu.VMEM((2, 128), jnp.int32),
        pltpu.VMEM((ROWS_PER_TILE, D), jnp.float32),
        pltpu.VMEM((TAILA, D), jnp.float32),
        pltpu.VMEM((TAILB, D), jnp.float32),
        pltpu.VMEM((1, TAILA), jnp.int32),
        pltpu.VMEM((1, TAILB), jnp.int32),
        pltpu.VMEM_SHARED((G, D), jnp.float32),
        pltpu.SemaphoreType.DMA,
        pltpu.SemaphoreType.DMA,
        pltpu.SemaphoreType.DMA,
        pltpu.SemaphoreType.DMA,
    ],
  )


BLKN = 1024
NBLK = (N + BLKN - 1) // BLKN
NPAD = NBLK * BLKN


def _counts_body(ids_ref, cnt_ref):
    i = pl.program_id(0)
    ids = ids_ref[0]
    hi = lax.shift_right_logical(ids, 7)
    lo = lax.bitwise_and(ids, 127)
    eh = (hi == lax.broadcasted_iota(jnp.int32, (1, 8), 1)
          ).astype(jnp.float32)
    el = (lo == lax.broadcasted_iota(jnp.int32, (1, 128), 1)
          ).astype(jnp.float32)
    hist = lax.dot_general(eh, el, (((0,), (0,)), ((), ())),
                           preferred_element_type=jnp.float32)

    @pl.when(i == 0)
    def _init():
        cnt_ref[...] = jnp.zeros((8, 128), jnp.float32)

    cnt_ref[...] += hist


_counts = pl.pallas_call(
    _counts_body,
    grid=(NBLK,),
    in_specs=[pl.BlockSpec((1, BLKN, 1), lambda i: (i, 0, 0))],
    out_specs=pl.BlockSpec((8, 128), lambda i: (0, 0)),
    out_shape=jax.ShapeDtypeStruct((8, 128), jnp.float32),
)


def _dense_body(sums_ref, cnt_ref, ds_ref, Ws_ref, bs_ref, Wh_ref, bh_ref,
                head_ref, var_ref):
    sums = sums_ref[0] + sums_ref[1]
    counts = cnt_ref[...]
    xg = sums / jnp.maximum(counts, 1.0)
    ds = ds_ref[...]

    out = jnp.zeros((G, 2 * HEAD_DIM), jnp.float32)
    for b in range(B):
        h = jnp.dot(xg, Ws_ref[b], preferred_element_type=jnp.float32)
        h = jnp.maximum(h + bs_ref[b][None, :], 0.0)
        o = jnp.dot(h, Wh_ref[b], preferred_element_type=jnp.float32)
        o = o + bh_ref[b][None, :]
        out = jnp.where(ds == b, o, out)

    head_ref[...] = out[:, :HEAD_DIM]
    var_ref[...] = out[:, HEAD_DIM:] ** 2


_dense = pl.pallas_call(
    _dense_body,
    out_shape=(
        jax.ShapeDtypeStruct((G, HEAD_DIM), jnp.float32),
        jax.ShapeDtypeStruct((G, HEAD_DIM), jnp.float32),
    ),
)


@jax.jit
def kernel(x, batch, dataset_name, W_shared, b_shared, W_head, b_head):
    sums = _get_pool()(x, batch)
    ids_pad = jnp.concatenate(
        [batch, jnp.full((NPAD - N,), G, jnp.int32)]).reshape(NBLK, BLKN, 1)
    cnt8 = _counts(ids_pad)
    counts = cnt8.reshape(G, 1)
    head, var = _dense(sums, counts, dataset_name, W_shared, b_shared,
                       W_head, b_head)
    return (head, var)


# speedup vs baseline: 1.0868x; 1.0868x over previous
import functools

import jax
import jax.numpy as jnp
from jax import lax
from jax.experimental import pallas as pl
from jax.experimental.pallas import tpu as pltpu
from jax.experimental.pallas import tpu_sc as plsc

N = 100000
D = 128
G = 1024
B = 8
H = 128
HEAD_DIM = 1

NC = 2
NS = 16
NW = NC * NS
CH = 256
NFULL = N // CH
TAILA = 128
TAILB = 32
ROWS_PER_TILE = G // NS


def _pool_body(x_hbm, batch_hbm, sums_hbm,
               buf0, buf1, idx0, idx1, zbuf, tbufa, tbufb, tidxa, tidxb,
               acc_sh, sem0, sem1, isem0, isem1):
    c = lax.axis_index("c")
    s = lax.axis_index("s")
    wid = s * NC + c

    def _fill(r, _):
        for k in range(D // 16):
            zbuf[r, pl.ds(k * 16, 16)] = jnp.zeros((16,), jnp.float32)
        return 0

    lax.fori_loop(0, ROWS_PER_TILE, _fill, 0)

    pltpu.sync_copy(zbuf, acc_sh.at[pl.ds(s * ROWS_PER_TILE, ROWS_PER_TILE)])
    plsc.subcore_barrier()

    nj = (NFULL - wid + NW - 1) // NW
    bufs = ((buf0, idx0, sem0, isem0), (buf1, idx1, sem1, isem1))

    def _issue(j, bufp, idxp, semp, isemp):
        base = (wid + j * NW) * CH
        pltpu.async_copy(x_hbm.at[pl.ds(base, CH)], bufp, semp)
        pltpu.async_copy(batch_hbm.at[pl.ds(base, 128)], idxp.at[0], isemp)
        pltpu.async_copy(batch_hbm.at[pl.ds(base + 128, 128)], idxp.at[1],
                         isemp)

    @pl.when(nj > 0)
    def _prologue():
        _issue(0, buf0, idx0, sem0, isem0)

    def _outer(j2, _):
        for b2 in (0, 1):
            j = j2 * 2 + b2
            bufp, idxp, semp, isemp = bufs[b2]
            bufn, idxn, semn, isemn = bufs[1 - b2]

            @pl.when(j < nj)
            def _do():
                @pl.when(j + 1 < nj)
                def _next():
                    _issue(j + 1, bufn, idxn, semn, isemn)

                base = (wid + j * NW) * CH
                pltpu.make_async_copy(x_hbm.at[pl.ds(base, CH)], bufp,
                                      semp).wait()
                pltpu.make_async_copy(batch_hbm.at[pl.ds(base, 128)],
                                      idxp.at[0], isemp).wait()
                pltpu.make_async_copy(batch_hbm.at[pl.ds(base, 128)],
                                      idxp.at[1], isemp).wait()
                for h in (0, 1):
                    pltpu.sync_copy(bufp.at[pl.ds(h * 128, 128)],
                                    acc_sh.at[idxp.at[h]], add=True)

        return 0

    lax.fori_loop(0, (nj + 1) // 2, _outer, 0)

    @pl.when(wid == NW - 1)
    def _tail():
        base = NFULL * CH
        pltpu.sync_copy(batch_hbm.at[pl.ds(base, TAILA)], tidxa.at[0])
        pltpu.sync_copy(x_hbm.at[pl.ds(base, TAILA)], tbufa)
        pltpu.sync_copy(tbufa, acc_sh.at[tidxa.at[0]], add=True)
        pltpu.sync_copy(batch_hbm.at[pl.ds(base + TAILA, TAILB)], tidxb.at[0])
        pltpu.sync_copy(x_hbm.at[pl.ds(base + TAILA, TAILB)], tbufb)
        pltpu.sync_copy(tbufb, acc_sh.at[tidxb.at[0]], add=True)

    plsc.subcore_barrier()

    pltpu.sync_copy(acc_sh.at[pl.ds(s * ROWS_PER_TILE, ROWS_PER_TILE)],
                    sums_hbm.at[c, pl.ds(s * ROWS_PER_TILE, ROWS_PER_TILE)])


@functools.lru_cache(maxsize=1)
def _get_pool():
  return pl.kernel(
    _pool_body,
    out_type=jax.ShapeDtypeStruct((NC, G, D), jnp.float32),
    mesh=plsc.VectorSubcoreMesh(core_axis_name="c", subcore_axis_name="s",
                                num_cores=NC, num_subcores=NS),
    scratch_types=[
        pltpu.VMEM((CH, D), jnp.float32),
        pltpu.VMEM((CH, D), jnp.float32),
        pltpu.VMEM((2, 128), jnp.int32),
        pltpu.VMEM((2, 128), jnp.int32),
        pltpu.VMEM((ROWS_PER_TILE, D), jnp.float32),
        pltpu.VMEM((TAILA, D), jnp.float32),
        pltpu.VMEM((TAILB, D), jnp.float32),
        pltpu.VMEM((1, TAILA), jnp.int32),
        pltpu.VMEM((1, TAILB), jnp.int32),
        pltpu.VMEM_SHARED((G, D), jnp.float32),
        pltpu.SemaphoreType.DMA,
        pltpu.SemaphoreType.DMA,
        pltpu.SemaphoreType.DMA,
        pltpu.SemaphoreType.DMA,
    ],
  )


BLKN = 1024
NBLK = (N + BLKN - 1) // BLKN
NPAD = NBLK * BLKN


def _counts_body(ids_ref, cnt_ref):
    def _blk(k, hist):
        ids = ids_ref[k]
        hi = lax.shift_right_logical(ids, 7)
        lo = lax.bitwise_and(ids, 127)
        eh = (hi == lax.broadcasted_iota(jnp.int32, (1, 8), 1)
              ).astype(jnp.float32)
        el = (lo == lax.broadcasted_iota(jnp.int32, (1, 128), 1)
              ).astype(jnp.float32)
        return hist + lax.dot_general(eh, el, (((0,), (0,)), ((), ())),
                                      preferred_element_type=jnp.float32)

    cnt_ref[...] = lax.fori_loop(0, NBLK, _blk,
                                 jnp.zeros((8, 128), jnp.float32))


_counts = pl.pallas_call(
    _counts_body,
    out_shape=jax.ShapeDtypeStruct((8, 128), jnp.float32),
)


def _dense_body(sums_ref, cnt_ref, ds_ref, Ws_ref, bs_ref, Wh_ref, bh_ref,
                head_ref, var_ref):
    sums = sums_ref[0] + sums_ref[1]
    counts = cnt_ref[...]
    xg = sums / jnp.maximum(counts, 1.0)
    ds = ds_ref[...]

    out = jnp.zeros((G, 2 * HEAD_DIM), jnp.float32)
    for b in range(B):
        h = jnp.dot(xg, Ws_ref[b], preferred_element_type=jnp.float32)
        h = jnp.maximum(h + bs_ref[b][None, :], 0.0)
        o = jnp.dot(h, Wh_ref[b], preferred_element_type=jnp.float32)
        o = o + bh_ref[b][None, :]
        out = jnp.where(ds == b, o, out)

    head_ref[...] = out[:, :HEAD_DIM]
    var_ref[...] = out[:, HEAD_DIM:] ** 2


_dense = pl.pallas_call(
    _dense_body,
    out_shape=(
        jax.ShapeDtypeStruct((G, HEAD_DIM), jnp.float32),
        jax.ShapeDtypeStruct((G, HEAD_DIM), jnp.float32),
    ),
)


@jax.jit
def kernel(x, batch, dataset_name, W_shared, b_shared, W_head, b_head):
    ids_pad = jnp.concatenate(
        [batch, jnp.full((NPAD - N,), G, jnp.int32)]).reshape(NBLK, BLKN, 1)
    cnt8 = _counts(ids_pad)
    sums = _get_pool()(x, batch)
    counts = cnt8.reshape(G, 1)
    head, var = _dense(sums, counts, dataset_name, W_shared, b_shared,
                       W_head, b_head)
    return (head, var)


# speedup vs baseline: 2.0544x; 1.8903x over previous
import functools

import jax
import jax.numpy as jnp
from jax import lax
from jax.experimental import pallas as pl
from jax.experimental.pallas import tpu as pltpu
from jax.experimental.pallas import tpu_sc as plsc

N = 100000
D = 128
G = 1024
B = 8
H = 128
HEAD_DIM = 1

NC = 2
NS = 16
NW = NC * NS
CH = 256
NFULL = N // CH
TAILA = 128
TAILB = 32
ROWS_PER_TILE = G // NS


def _pool_body(x_hbm, batch_hbm, sums_hbm, cnt_hbm,
               buf0, buf1, idx0, idx1, ones, zbuf, tbufa, tbufb, tidxa, tidxb,
               acc_sh, cnt_sh, sem0, sem1, isem0, isem1, ssem0, ssem1):
    c = lax.axis_index("c")
    s = lax.axis_index("s")
    wid = s * NC + c

    def _fill(r, _):
        for k in range(D // 16):
            ones[r, pl.ds(k * 16, 16)] = jnp.full((16,), 1.0, jnp.float32)

        @pl.when(r < ROWS_PER_TILE)
        def _z():
            for k in range(D // 16):
                zbuf[r, pl.ds(k * 16, 16)] = jnp.zeros((16,), jnp.float32)

        return 0

    lax.fori_loop(0, 128, _fill, 0)

    pltpu.sync_copy(zbuf, acc_sh.at[pl.ds(s * ROWS_PER_TILE, ROWS_PER_TILE)])
    pltpu.sync_copy(zbuf, cnt_sh.at[pl.ds(s * ROWS_PER_TILE, ROWS_PER_TILE)])
    plsc.subcore_barrier()

    nj = (NFULL - wid + NW - 1) // NW
    bufs = ((buf0, idx0, sem0, isem0, ssem0), (buf1, idx1, sem1, isem1, ssem1))

    def _drain_scatter(bufp, idxp, ssemp):
        for h in (0, 1):
            pltpu.make_async_copy(bufp.at[pl.ds(h * 128, 128)],
                                  acc_sh.at[idxp.at[h]], ssemp).wait()
            pltpu.make_async_copy(ones, cnt_sh.at[idxp.at[h]], ssemp).wait()

    def _issue(j, bufp, idxp, semp, isemp):
        base = (wid + j * NW) * CH
        pltpu.async_copy(x_hbm.at[pl.ds(base, CH)], bufp, semp)
        pltpu.async_copy(batch_hbm.at[pl.ds(base, 128)], idxp.at[0], isemp)
        pltpu.async_copy(batch_hbm.at[pl.ds(base + 128, 128)], idxp.at[1],
                         isemp)

    @pl.when(nj > 0)
    def _prologue():
        _issue(0, buf0, idx0, sem0, isem0)

    def _outer(j2, _):
        for b2 in (0, 1):
            j = j2 * 2 + b2
            bufp, idxp, semp, isemp, ssemp = bufs[b2]
            bufn, idxn, semn, isemn, ssemn = bufs[1 - b2]

            @pl.when(j < nj)
            def _do():
                @pl.when(j >= 1)
                def _drain_prev():
                    _drain_scatter(bufn, idxn, ssemn)

                @pl.when(j + 1 < nj)
                def _next():
                    _issue(j + 1, bufn, idxn, semn, isemn)

                base = (wid + j * NW) * CH
                pltpu.make_async_copy(x_hbm.at[pl.ds(base, CH)], bufp,
                                      semp).wait()
                pltpu.make_async_copy(batch_hbm.at[pl.ds(base, 128)],
                                      idxp.at[0], isemp).wait()
                pltpu.make_async_copy(batch_hbm.at[pl.ds(base, 128)],
                                      idxp.at[1], isemp).wait()
                for h in (0, 1):
                    pltpu.async_copy(bufp.at[pl.ds(h * 128, 128)],
                                     acc_sh.at[idxp.at[h]], ssemp, add=True)
                    pltpu.async_copy(ones, cnt_sh.at[idxp.at[h]], ssemp,
                                     add=True)

        return 0

    lax.fori_loop(0, (nj + 1) // 2, _outer, 0)

    for b2 in (0, 1):
        bufp, idxp, _semp, _isemp, ssemp = bufs[b2]

        @pl.when((nj >= 1) & (lax.rem(nj - 1, 2) == b2))
        def _drain_last():
            _drain_scatter(bufp, idxp, ssemp)

    @pl.when(wid == NW - 1)
    def _tail():
        base = NFULL * CH
        pltpu.sync_copy(batch_hbm.at[pl.ds(base, TAILA)], tidxa.at[0])
        pltpu.sync_copy(x_hbm.at[pl.ds(base, TAILA)], tbufa)
        pltpu.sync_copy(tbufa, acc_sh.at[tidxa.at[0]], add=True)
        pltpu.sync_copy(ones, cnt_sh.at[tidxa.at[0]], add=True)
        pltpu.sync_copy(batch_hbm.at[pl.ds(base + TAILA, TAILB)], tidxb.at[0])
        pltpu.sync_copy(x_hbm.at[pl.ds(base + TAILA, TAILB)], tbufb)
        pltpu.sync_copy(tbufb, acc_sh.at[tidxb.at[0]], add=True)
        pltpu.sync_copy(ones.at[pl.ds(0, TAILB)], cnt_sh.at[tidxb.at[0]],
                        add=True)

    plsc.subcore_barrier()

    pltpu.sync_copy(acc_sh.at[pl.ds(s * ROWS_PER_TILE, ROWS_PER_TILE)],
                    sums_hbm.at[c, pl.ds(s * ROWS_PER_TILE, ROWS_PER_TILE)])
    pltpu.sync_copy(cnt_sh.at[pl.ds(s * ROWS_PER_TILE, ROWS_PER_TILE)],
                    cnt_hbm.at[c, pl.ds(s * ROWS_PER_TILE, ROWS_PER_TILE)])


@functools.lru_cache(maxsize=1)
def _get_pool():
  return pl.kernel(
    _pool_body,
    out_type=(
        jax.ShapeDtypeStruct((NC, G, D), jnp.float32),
        jax.ShapeDtypeStruct((NC, G, D), jnp.float32),
    ),
    mesh=plsc.VectorSubcoreMesh(core_axis_name="c", subcore_axis_name="s",
                                num_cores=NC, num_subcores=NS),
    scratch_types=[
        pltpu.VMEM((CH, D), jnp.float32),
        pltpu.VMEM((CH, D), jnp.float32),
        pltpu.VMEM((2, 128), jnp.int32),
        pltpu.VMEM((2, 128), jnp.int32),
        pltpu.VMEM((128, D), jnp.float32),
        pltpu.VMEM((ROWS_PER_TILE, D), jnp.float32),
        pltpu.VMEM((TAILA, D), jnp.float32),
        pltpu.VMEM((TAILB, D), jnp.float32),
        pltpu.VMEM((1, TAILA), jnp.int32),
        pltpu.VMEM((1, TAILB), jnp.int32),
        pltpu.VMEM_SHARED((G, D), jnp.float32),
        pltpu.VMEM_SHARED((G, D), jnp.float32),
        pltpu.SemaphoreType.DMA,
        pltpu.SemaphoreType.DMA,
        pltpu.SemaphoreType.DMA,
        pltpu.SemaphoreType.DMA,
        pltpu.SemaphoreType.DMA,
        pltpu.SemaphoreType.DMA,
    ],
  )


def _dense_body(sums_ref, cnt_ref, ds_ref, Ws_ref, bs_ref, Wh_ref, bh_ref,
                head_ref, var_ref):
    sums = sums_ref[0] + sums_ref[1]
    counts = cnt_ref[0, :, 0:1] + cnt_ref[1, :, 0:1]
    xg = sums / jnp.maximum(counts, 1.0)
    ds = ds_ref[...]

    out = jnp.zeros((G, 2 * HEAD_DIM), jnp.float32)
    for b in range(B):
        h = jnp.dot(xg, Ws_ref[b], preferred_element_type=jnp.float32)
        h = jnp.maximum(h + bs_ref[b][None, :], 0.0)
        o = jnp.dot(h, Wh_ref[b], preferred_element_type=jnp.float32)
        o = o + bh_ref[b][None, :]
        out = jnp.where(ds == b, o, out)

    head_ref[...] = out[:, :HEAD_DIM]
    var_ref[...] = out[:, HEAD_DIM:] ** 2


_dense = pl.pallas_call(
    _dense_body,
    out_shape=(
        jax.ShapeDtypeStruct((G, HEAD_DIM), jnp.float32),
        jax.ShapeDtypeStruct((G, HEAD_DIM), jnp.float32),
    ),
)


@jax.jit
def kernel(x, batch, dataset_name, W_shared, b_shared, W_head, b_head):
    sums, cnt = _get_pool()(x, batch)
    head, var = _dense(sums, cnt, dataset_name, W_shared, b_shared,
                       W_head, b_head)
    return (head, var)
